# Initial kernel scaffold; baseline (speedup 1.0000x reference)
import jax
import jax.numpy as jnp
from jax.experimental import pallas as pl


def kernel(x, edge_index, batch, W1, b1, W2, b2, lin_W, lin_b):
    raise NotImplementedError("write your pallas kernel here")


# speedup vs baseline: 15.8019x; 15.8019x over previous
import functools

import jax
import jax.numpy as jnp
from jax import lax
from jax.experimental import pallas as pl
from jax.experimental.pallas import tpu as pltpu
from jax.experimental.pallas import tpu_sc as plsc

N = 100000
E = 1600000
IN = 32
H = 64
G = 64

L = 16
NTILES = 16
BN = 2048
NP = 49 * BN
EP = 1638400
EROWS = EP // 128
CH = 1280
KSUB = CH // 128
RPT = NP // NTILES
STG = RPT // 16

_f32 = jnp.float32
_i32 = jnp.int32


def _fill(ref, rows, vec):
    def body(i, _):
        ref[i] = vec
        return 0
    lax.fori_loop(0, rows, body, 0)


def _zero_accum(stage, accum, s):
    _fill(stage, STG, jnp.zeros((L,), _f32))
    rz = s * RPT
    for r in range(16):
        pltpu.sync_copy(stage, accum.at[pl.ds(rz + r * STG, STG)])


def _write_out(stage, accum, out, s):
    rz = s * RPT
    for r in range(16):
        pltpu.sync_copy(accum.at[pl.ds(rz + r * STG, STG)], stage)
        pltpu.sync_copy(stage, out.at[pl.ds(rz + r * STG, STG)])


def _scan_edges(src2, dst2, table, sidx, didx, rows, accum, semg, sems, s):
    def chunk(j, _):
        rb = s * (EP // NTILES // 128) + j * KSUB
        pltpu.sync_copy(src2.at[pl.ds(rb, KSUB)], sidx)
        pltpu.sync_copy(dst2.at[pl.ds(rb, KSUB)], didx)
        gd = [pltpu.async_copy(table.at[sidx.at[k]],
                               rows.at[pl.ds(k * 128, 128)], semg)
              for k in range(KSUB)]
        sd = []
        for k in range(KSUB):
            gd[k].wait()
            sd.append(pltpu.async_copy(rows.at[pl.ds(k * 128, 128)],
                                       accum.at[didx.at[k]], sems, add=True))
        for d in sd:
            d.wait()
        return 0
    lax.fori_loop(0, EP // NTILES // CH, chunk, 0)


def _agg_pass(src2, dst2, table, out, sidx, didx, rows, stage, accum,
              semg, sems, s):
    _zero_accum(stage, accum, s)
    plsc.subcore_barrier()
    _scan_edges(src2, dst2, table, sidx, didx, rows, accum, semg, sems, s)
    plsc.subcore_barrier()
    _write_out(stage, accum, out, s)
    plsc.subcore_barrier()


@functools.partial(
    pl.kernel,
    out_type=jax.ShapeDtypeStruct((NP, L), _f32),
    mesh=plsc.VectorSubcoreMesh(core_axis_name="c", subcore_axis_name="s", num_cores=1),
    scratch_types=[
        pltpu.VMEM((KSUB, 128), _i32),
        pltpu.VMEM((128, L), _f32),
        pltpu.VMEM((STG, L), _f32),
        pltpu.VMEM_SHARED((NP, L), _f32),
        pltpu.SemaphoreType.DMA,
    ],
    compiler_params=pltpu.CompilerParams(use_tc_tiling_on_sc=False),
)
def _deg_kernel(dst2, o0, didx, ones_v, stage, accum, sem):
    s = lax.axis_index("s")
    _fill(ones_v, 128, jnp.ones((L,), _f32))
    _zero_accum(stage, accum, s)
    plsc.subcore_barrier()

    def chunk(j, _):
        rb = s * (EP // NTILES // 128) + j * KSUB
        pltpu.sync_copy(dst2.at[pl.ds(rb, KSUB)], didx)
        sd = [pltpu.async_copy(ones_v, accum.at[didx.at[k]], sem, add=True)
              for k in range(KSUB)]
        for d in sd:
            d.wait()
        return 0
    lax.fori_loop(0, EP // NTILES // CH, chunk, 0)
    plsc.subcore_barrier()
    _write_out(stage, accum, o0, s)


@functools.partial(
    pl.kernel,
    out_type=(jax.ShapeDtypeStruct((NP, L), _f32),
              jax.ShapeDtypeStruct((NP, L), _f32)),
    mesh=plsc.VectorSubcoreMesh(core_axis_name="c", subcore_axis_name="s", num_cores=1),
    scratch_types=[
        pltpu.VMEM((KSUB, 128), _i32),
        pltpu.VMEM((KSUB, 128), _i32),
        pltpu.VMEM((CH, L), _f32),
        pltpu.VMEM((STG, L), _f32),
        pltpu.VMEM_SHARED((NP, L), _f32),
        pltpu.SemaphoreType.DMA,
        pltpu.SemaphoreType.DMA,
    ],
    compiler_params=pltpu.CompilerParams(use_tc_tiling_on_sc=False),
)
def _agg1_kernel(src2, dst2, xs0, xs1, t0, t1,
                 sidx, didx, rows, stage, accum, semg, sems):
    s = lax.axis_index("s")
    _agg_pass(src2, dst2, xs0, t0, sidx, didx, rows, stage, accum,
              semg, sems, s)
    _agg_pass(src2, dst2, xs1, t1, sidx, didx, rows, stage, accum,
              semg, sems, s)


@functools.partial(
    pl.kernel,
    out_type=tuple(jax.ShapeDtypeStruct((NP, L), _f32) for _ in range(4)),
    mesh=plsc.VectorSubcoreMesh(core_axis_name="c", subcore_axis_name="s", num_cores=1),
    scratch_types=[
        pltpu.VMEM((KSUB, 128), _i32),
        pltpu.VMEM((KSUB, 128), _i32),
        pltpu.VMEM((CH, L), _f32),
        pltpu.VMEM((STG, L), _f32),
        pltpu.VMEM_SHARED((NP, L), _f32),
        pltpu.SemaphoreType.DMA,
        pltpu.SemaphoreType.DMA,
    ],
    compiler_params=pltpu.CompilerParams(use_tc_tiling_on_sc=False),
)
def _agg2_kernel(src2, dst2, g0, g1, g2, g3, u0, u1, u2, u3,
                 sidx, didx, rows, stage, accum, semg, sems):
    s = lax.axis_index("s")
    for tbl, out in ((g0, u0), (g1, u1), (g2, u2), (g3, u3)):
        _agg_pass(src2, dst2, tbl, out, sidx, didx, rows, stage, accum,
                  semg, sems, s)


def _prep_body(p0, x, dinv_o, xs0_o, xs1_o):
    deg = 1.0 + p0[:, 0:1]
    dinv = lax.rsqrt(deg)
    xs = x[...] * dinv
    dinv_o[...] = dinv
    xs0_o[...] = xs[:, :L]
    xs1_o[...] = xs[:, L:]


def _layer1_body(t0, t1, xs0, xs1, dinv, w1, b1, g0, g1, g2, g3):
    agg = jnp.concatenate([t0[...] + xs0[...], t1[...] + xs1[...]], axis=1)
    agg = agg * dinv[...]
    h = jnp.dot(agg, w1[...], preferred_element_type=_f32) + b1[...]
    g = jnp.maximum(h, 0.0) * dinv[...]
    g0[...] = g[:, 0:L]
    g1[...] = g[:, L:2 * L]
    g2[...] = g[:, 2 * L:3 * L]
    g3[...] = g[:, 3 * L:]


def _layer2_body(u0, u1, u2, u3, g0, g1, g2, g3, dinv, batch, w2, b2,
                 lw, lb, out_o, s_acc, c_acc):
    i = pl.program_id(0)
    agg = jnp.concatenate([u0[...] + g0[...], u1[...] + g1[...],
                           u2[...] + g2[...], u3[...] + g3[...]], axis=1)
    agg = agg * dinv[...]
    h = jnp.maximum(jnp.dot(agg, w2[...], preferred_element_type=_f32)
                    + b2[...], 0.0)
    z = jnp.dot(h, lw[...], preferred_element_type=_f32)
    rowid = lax.broadcasted_iota(_i32, (BN, 1), 0) + i * BN
    valid = rowid < N
    zm = jnp.where(valid, z, 0.0)
    seg = lax.broadcasted_iota(_i32, (BN, G), 1)
    oh = jnp.where(valid & (batch[...] == seg), 1.0, 0.0)
    psum = jnp.sum(zm * oh, axis=0, keepdims=True)
    pcnt = jnp.sum(oh, axis=0, keepdims=True)

    @pl.when(i == 0)
    def _():
        s_acc[...] = jnp.zeros_like(s_acc)
        c_acc[...] = jnp.zeros_like(c_acc)

    s_acc[...] += psum
    c_acc[...] += pcnt

    @pl.when(i == pl.num_programs(0) - 1)
    def _():
        out_o[...] = s_acc[...] / jnp.maximum(c_acc[...], 1.0) + lb[...]


def _node_spec(width):
    return pl.BlockSpec((BN, width), lambda i: (i, 0))


def _const_spec(shape):
    return pl.BlockSpec(shape, lambda i: (0, 0))


def kernel(x, edge_index, batch, W1, b1, W2, b2, lin_W, lin_b):
    src = edge_index[0]
    dst = edge_index[1]
    pad = jnp.arange(EP - E, dtype=_i32) % 128
    src2 = jnp.concatenate([src, pad]).reshape(EROWS, 128)
    dst2 = jnp.concatenate([dst, N + pad]).reshape(EROWS, 128)

    p0 = _deg_kernel(dst2)

    grid = (NP // BN,)
    dinv, xs0, xs1 = pl.pallas_call(
        _prep_body,
        grid=grid,
        in_specs=[_node_spec(L), _node_spec(IN)],
        out_specs=[_node_spec(1), _node_spec(L), _node_spec(L)],
        out_shape=[jax.ShapeDtypeStruct((NP, 1), _f32),
                   jax.ShapeDtypeStruct((NP, L), _f32),
                   jax.ShapeDtypeStruct((NP, L), _f32)],
    )(p0, x)

    t0, t1 = _agg1_kernel(src2, dst2, xs0, xs1)

    g0, g1, g2, g3 = pl.pallas_call(
        _layer1_body,
        grid=grid,
        in_specs=[_node_spec(L)] * 4 + [_node_spec(1),
                  _const_spec((IN, H)), _const_spec((1, H))],
        out_specs=[_node_spec(L)] * 4,
        out_shape=[jax.ShapeDtypeStruct((NP, L), _f32) for _ in range(4)],
    )(t0, t1, xs0, xs1, dinv, W1, b1.reshape(1, H))

    u0, u1, u2, u3 = _agg2_kernel(src2, dst2, g0, g1, g2, g3)

    out = pl.pallas_call(
        _layer2_body,
        grid=grid,
        in_specs=[_node_spec(L)] * 8 + [_node_spec(1), _node_spec(1),
                  _const_spec((H, H)), _const_spec((1, H)),
                  _const_spec((H, 1)), _const_spec((1, 1))],
        out_specs=pl.BlockSpec((1, G), lambda i: (0, 0)),
        out_shape=jax.ShapeDtypeStruct((1, G), _f32),
        scratch_shapes=[pltpu.VMEM((1, G), _f32), pltpu.VMEM((1, G), _f32)],
    )(u0, u1, u2, u3, g0, g1, g2, g3, dinv, batch.reshape(N, 1),
      W2, b2.reshape(1, H), lin_W, lin_b.reshape(1, 1))

    return out.reshape(G, 1)


# speedup vs baseline: 22.9572x; 1.4528x over previous
import functools

import jax
import jax.numpy as jnp
from jax import lax
from jax.experimental import pallas as pl
from jax.experimental.pallas import tpu as pltpu
from jax.experimental.pallas import tpu_sc as plsc

N = 100000
E = 1600000
IN = 32
H = 64
G = 64

L = 16
NTILES = 16
BN = 2048
NP = 49 * BN
EP = 1638400
EROWS = EP // 128
CH = 1280
KSUB = CH // 128
RPT = NP // NTILES
STG = RPT // 16

_f32 = jnp.float32
_i32 = jnp.int32


def _fill(ref, rows, vec):
    def body(i, _):
        ref[i] = vec
        return 0
    lax.fori_loop(0, rows, body, 0)


def _zero_accum(stage, accum, s):
    _fill(stage, STG, jnp.zeros((L,), _f32))
    rz = s * RPT
    for r in range(16):
        pltpu.sync_copy(stage, accum.at[pl.ds(rz + r * STG, STG)])


def _write_out(stage, accum, out, s):
    rz = s * RPT
    for r in range(16):
        pltpu.sync_copy(accum.at[pl.ds(rz + r * STG, STG)], stage)
        pltpu.sync_copy(stage, out.at[pl.ds(rz + r * STG, STG)])


def _scan_edges(src2, dst2, table, sidx, didx, rows, accum, semg, sems, s):
    def chunk(j, _):
        rb = s * (EP // NTILES // 128) + j * KSUB
        pltpu.sync_copy(src2.at[pl.ds(rb, KSUB)], sidx)
        pltpu.sync_copy(dst2.at[pl.ds(rb, KSUB)], didx)
        gd = [pltpu.async_copy(table.at[sidx.at[k]],
                               rows.at[pl.ds(k * 128, 128)], semg)
              for k in range(KSUB)]
        sd = []
        for k in range(KSUB):
            gd[k].wait()
            sd.append(pltpu.async_copy(rows.at[pl.ds(k * 128, 128)],
                                       accum.at[didx.at[k]], sems, add=True))
        for d in sd:
            d.wait()
        return 0
    lax.fori_loop(0, EP // NTILES // CH, chunk, 0)


def _agg_pass(src2, dst2, table, out, sidx, didx, rows, stage, accum,
              semg, sems, s):
    _zero_accum(stage, accum, s)
    plsc.subcore_barrier()
    _scan_edges(src2, dst2, table, sidx, didx, rows, accum, semg, sems, s)
    plsc.subcore_barrier()
    _write_out(stage, accum, out, s)
    plsc.subcore_barrier()


@functools.partial(
    pl.kernel,
    out_type=(jax.ShapeDtypeStruct((NP, L), _f32),
              jax.ShapeDtypeStruct((NP, L), _f32)),
    mesh=plsc.VectorSubcoreMesh(core_axis_name="c", subcore_axis_name="s"),
    scratch_types=[
        pltpu.VMEM((KSUB, 128), _i32),
        pltpu.VMEM((128, L), _f32),
        pltpu.VMEM((STG, L), _f32),
        pltpu.VMEM_SHARED((NP, L), _f32),
        pltpu.SemaphoreType.DMA,
    ],
    compiler_params=pltpu.CompilerParams(use_tc_tiling_on_sc=False),
)
def _deg_kernel(dst2, o0, o1, didx, ones_v, stage, accum, sem):
    c = lax.axis_index("c")
    s = lax.axis_index("s")
    w = c * NTILES + s
    _fill(ones_v, 128, jnp.ones((L,), _f32))
    _zero_accum(stage, accum, s)
    plsc.subcore_barrier()

    def chunk(j, _):
        rb = w * (EP // 32 // 128) + j * KSUB
        pltpu.sync_copy(dst2.at[pl.ds(rb, KSUB)], didx)
        sd = [pltpu.async_copy(ones_v, accum.at[didx.at[k]], sem, add=True)
              for k in range(KSUB)]
        for d in sd:
            d.wait()
        return 0
    lax.fori_loop(0, EP // 32 // CH, chunk, 0)
    plsc.subcore_barrier()

    @pl.when(c == 0)
    def _():
        _write_out(stage, accum, o0, s)

    @pl.when(c == 1)
    def _():
        _write_out(stage, accum, o1, s)


@functools.partial(
    pl.kernel,
    out_type=(jax.ShapeDtypeStruct((NP, L), _f32),
              jax.ShapeDtypeStruct((NP, L), _f32)),
    mesh=plsc.VectorSubcoreMesh(core_axis_name="c", subcore_axis_name="s"),
    scratch_types=[
        pltpu.VMEM((KSUB, 128), _i32),
        pltpu.VMEM((KSUB, 128), _i32),
        pltpu.VMEM((CH, L), _f32),
        pltpu.VMEM((STG, L), _f32),
        pltpu.VMEM_SHARED((NP, L), _f32),
        pltpu.SemaphoreType.DMA,
        pltpu.SemaphoreType.DMA,
    ],
    compiler_params=pltpu.CompilerParams(use_tc_tiling_on_sc=False),
)
def _agg1_kernel(src2, dst2, xs0, xs1, t0, t1,
                 sidx, didx, rows, stage, accum, semg, sems):
    c = lax.axis_index("c")
    s = lax.axis_index("s")

    @pl.when(c == 0)
    def _():
        _agg_pass(src2, dst2, xs0, t0, sidx, didx, rows, stage, accum,
                  semg, sems, s)

    @pl.when(c == 1)
    def _():
        _agg_pass(src2, dst2, xs1, t1, sidx, didx, rows, stage, accum,
                  semg, sems, s)


@functools.partial(
    pl.kernel,
    out_type=tuple(jax.ShapeDtypeStruct((NP, L), _f32) for _ in range(4)),
    mesh=plsc.VectorSubcoreMesh(core_axis_name="c", subcore_axis_name="s"),
    scratch_types=[
        pltpu.VMEM((KSUB, 128), _i32),
        pltpu.VMEM((KSUB, 128), _i32),
        pltpu.VMEM((CH, L), _f32),
        pltpu.VMEM((STG, L), _f32),
        pltpu.VMEM_SHARED((NP, L), _f32),
        pltpu.SemaphoreType.DMA,
        pltpu.SemaphoreType.DMA,
    ],
    compiler_params=pltpu.CompilerParams(use_tc_tiling_on_sc=False),
)
def _agg2_kernel(src2, dst2, g0, g1, g2, g3, u0, u1, u2, u3,
                 sidx, didx, rows, stage, accum, semg, sems):
    c = lax.axis_index("c")
    s = lax.axis_index("s")

    @pl.when(c == 0)
    def _():
        for tbl, out in ((g0, u0), (g1, u1)):
            _agg_pass(src2, dst2, tbl, out, sidx, didx, rows, stage, accum,
                      semg, sems, s)

    @pl.when(c == 1)
    def _():
        for tbl, out in ((g2, u2), (g3, u3)):
            _agg_pass(src2, dst2, tbl, out, sidx, didx, rows, stage, accum,
                      semg, sems, s)


def _prep_body(p0, p1, x, dinv_o, xs0_o, xs1_o):
    deg = 1.0 + p0[:, 0:1] + p1[:, 0:1]
    dinv = lax.rsqrt(deg)
    xs = x[...] * dinv
    dinv_o[...] = dinv
    xs0_o[...] = xs[:, :L]
    xs1_o[...] = xs[:, L:]


def _layer1_body(t0, t1, xs0, xs1, dinv, w1, b1, g0, g1, g2, g3):
    agg = jnp.concatenate([t0[...] + xs0[...], t1[...] + xs1[...]], axis=1)
    agg = agg * dinv[...]
    h = jnp.dot(agg, w1[...], preferred_element_type=_f32) + b1[...]
    g = jnp.maximum(h, 0.0) * dinv[...]
    g0[...] = g[:, 0:L]
    g1[...] = g[:, L:2 * L]
    g2[...] = g[:, 2 * L:3 * L]
    g3[...] = g[:, 3 * L:]


def _layer2_body(u0, u1, u2, u3, g0, g1, g2, g3, dinv, batch, w2, b2,
                 lw, lb, out_o, s_acc, c_acc):
    i = pl.program_id(0)
    agg = jnp.concatenate([u0[...] + g0[...], u1[...] + g1[...],
                           u2[...] + g2[...], u3[...] + g3[...]], axis=1)
    agg = agg * dinv[...]
    h = jnp.maximum(jnp.dot(agg, w2[...], preferred_element_type=_f32)
                    + b2[...], 0.0)
    z = jnp.dot(h, lw[...], preferred_element_type=_f32)
    rowid = lax.broadcasted_iota(_i32, (BN, 1), 0) + i * BN
    valid = rowid < N
    zm = jnp.where(valid, z, 0.0)
    seg = lax.broadcasted_iota(_i32, (BN, G), 1)
    oh = jnp.where(valid & (batch[...] == seg), 1.0, 0.0)
    psum = jnp.sum(zm * oh, axis=0, keepdims=True)
    pcnt = jnp.sum(oh, axis=0, keepdims=True)

    @pl.when(i == 0)
    def _():
        s_acc[...] = jnp.zeros_like(s_acc)
        c_acc[...] = jnp.zeros_like(c_acc)

    s_acc[...] += psum
    c_acc[...] += pcnt

    @pl.when(i == pl.num_programs(0) - 1)
    def _():
        out_o[...] = s_acc[...] / jnp.maximum(c_acc[...], 1.0) + lb[...]


def _node_spec(width):
    return pl.BlockSpec((BN, width), lambda i: (i, 0))


def _const_spec(shape):
    return pl.BlockSpec(shape, lambda i: (0, 0))


def kernel(x, edge_index, batch, W1, b1, W2, b2, lin_W, lin_b):
    src = edge_index[0]
    dst = edge_index[1]
    pad = jnp.arange(EP - E, dtype=_i32) % 128
    src2 = jnp.concatenate([src, pad]).reshape(EROWS, 128)
    dst2 = jnp.concatenate([dst, N + pad]).reshape(EROWS, 128)

    p0, p1 = _deg_kernel(dst2)

    grid = (NP // BN,)
    dinv, xs0, xs1 = pl.pallas_call(
        _prep_body,
        grid=grid,
        in_specs=[_node_spec(L), _node_spec(L), _node_spec(IN)],
        out_specs=[_node_spec(1), _node_spec(L), _node_spec(L)],
        out_shape=[jax.ShapeDtypeStruct((NP, 1), _f32),
                   jax.ShapeDtypeStruct((NP, L), _f32),
                   jax.ShapeDtypeStruct((NP, L), _f32)],
    )(p0, p1, x)

    t0, t1 = _agg1_kernel(src2, dst2, xs0, xs1)

    g0, g1, g2, g3 = pl.pallas_call(
        _layer1_body,
        grid=grid,
        in_specs=[_node_spec(L)] * 4 + [_node_spec(1),
                  _const_spec((IN, H)), _const_spec((1, H))],
        out_specs=[_node_spec(L)] * 4,
        out_shape=[jax.ShapeDtypeStruct((NP, L), _f32) for _ in range(4)],
    )(t0, t1, xs0, xs1, dinv, W1, b1.reshape(1, H))

    u0, u1, u2, u3 = _agg2_kernel(src2, dst2, g0, g1, g2, g3)

    out = pl.pallas_call(
        _layer2_body,
        grid=grid,
        in_specs=[_node_spec(L)] * 8 + [_node_spec(1), _node_spec(1),
                  _const_spec((H, H)), _const_spec((1, H)),
                  _const_spec((H, 1)), _const_spec((1, 1))],
        out_specs=pl.BlockSpec((1, G), lambda i: (0, 0)),
        out_shape=jax.ShapeDtypeStruct((1, G), _f32),
        scratch_shapes=[pltpu.VMEM((1, G), _f32), pltpu.VMEM((1, G), _f32)],
    )(u0, u1, u2, u3, g0, g1, g2, g3, dinv, batch.reshape(N, 1),
      W2, b2.reshape(1, H), lin_W, lin_b.reshape(1, 1))

    return out.reshape(G, 1)


# speedup vs baseline: 34.3877x; 1.4979x over previous
import functools

import jax
import jax.numpy as jnp
from jax import lax
from jax.experimental import pallas as pl
from jax.experimental.pallas import tpu as pltpu
from jax.experimental.pallas import tpu_sc as plsc

N = 100000
E = 1600000
IN = 32
H = 64
G = 64

L = 16
NTILES = 16
BN = 2048
NP = 49 * BN
EP = 1638400
EROWS = EP // 128
CH = 1280
KSUB = CH // 128
RPT = NP // NTILES
STG = RPT // 16

_f32 = jnp.float32
_i32 = jnp.int32


def _fill(ref, rows, vec):
    def body(i, _):
        ref[i] = vec
        return 0
    lax.fori_loop(0, rows, body, 0)


def _zero_accum(stage, accum, s):
    _fill(stage, STG, jnp.zeros((L,), _f32))
    rz = s * RPT
    for r in range(16):
        pltpu.sync_copy(stage, accum.at[pl.ds(rz + r * STG, STG)])


def _write_out(stage, accum, out, s):
    rz = s * RPT
    for r in range(16):
        pltpu.sync_copy(accum.at[pl.ds(rz + r * STG, STG)], stage)
        pltpu.sync_copy(stage, out.at[pl.ds(rz + r * STG, STG)])


def _scan_edges(src2, dst2, table, sidx, didx, rows, accum, semg, sems, s):
    def chunk(j, _):
        rb = s * (EP // NTILES // 128) + j * KSUB
        pltpu.sync_copy(src2.at[pl.ds(rb, KSUB)], sidx)
        pltpu.sync_copy(dst2.at[pl.ds(rb, KSUB)], didx)
        gd = [pltpu.async_copy(table.at[sidx.at[k]],
                               rows.at[pl.ds(k * 128, 128)], semg)
              for k in range(KSUB)]
        sd = []
        for k in range(KSUB):
            gd[k].wait()
            sd.append(pltpu.async_copy(rows.at[pl.ds(k * 128, 128)],
                                       accum.at[didx.at[k]], sems, add=True))
        for d in sd:
            d.wait()
        return 0
    lax.fori_loop(0, EP // NTILES // CH, chunk, 0)


def _agg_pass(src2, dst2, table, out, sidx, didx, rows, stage, accum,
              semg, sems, s):
    _zero_accum(stage, accum, s)
    plsc.subcore_barrier()
    _scan_edges(src2, dst2, table, sidx, didx, rows, accum, semg, sems, s)
    plsc.subcore_barrier()
    _write_out(stage, accum, out, s)
    plsc.subcore_barrier()


@functools.partial(
    pl.kernel,
    out_type=(jax.ShapeDtypeStruct((NP, L), _f32),
              jax.ShapeDtypeStruct((NP, L), _f32)),
    mesh=plsc.VectorSubcoreMesh(core_axis_name="c", subcore_axis_name="s"),
    scratch_types=[
        pltpu.VMEM((KSUB, 128), _i32),
        pltpu.VMEM((128, L), _f32),
        pltpu.VMEM((STG, L), _f32),
        pltpu.VMEM_SHARED((NP, L), _f32),
        pltpu.SemaphoreType.DMA,
    ],
    compiler_params=pltpu.CompilerParams(use_tc_tiling_on_sc=False),
)
def _deg_kernel(dst2, o0, o1, didx, ones_v, stage, accum, sem):
    c = lax.axis_index("c")
    s = lax.axis_index("s")
    w = c * NTILES + s
    _fill(ones_v, 128, jnp.ones((L,), _f32))
    _zero_accum(stage, accum, s)
    plsc.subcore_barrier()

    def chunk(j, _):
        rb = w * (EP // 32 // 128) + j * KSUB
        pltpu.sync_copy(dst2.at[pl.ds(rb, KSUB)], didx)
        sd = [pltpu.async_copy(ones_v, accum.at[didx.at[k]], sem, add=True)
              for k in range(KSUB)]
        for d in sd:
            d.wait()
        return 0
    lax.fori_loop(0, EP // 32 // CH, chunk, 0)
    plsc.subcore_barrier()

    @pl.when(c == 0)
    def _():
        _write_out(stage, accum, o0, s)

    @pl.when(c == 1)
    def _():
        _write_out(stage, accum, o1, s)


@functools.partial(
    pl.kernel,
    out_type=(jax.ShapeDtypeStruct((NP, L), _f32),
              jax.ShapeDtypeStruct((NP, L), _f32)),
    mesh=plsc.VectorSubcoreMesh(core_axis_name="c", subcore_axis_name="s"),
    scratch_types=[
        pltpu.VMEM((KSUB, 128), _i32),
        pltpu.VMEM((KSUB, 128), _i32),
        pltpu.VMEM((CH, L), _f32),
        pltpu.VMEM((STG, L), _f32),
        pltpu.VMEM_SHARED((NP, L), _f32),
        pltpu.SemaphoreType.DMA,
        pltpu.SemaphoreType.DMA,
    ],
    compiler_params=pltpu.CompilerParams(use_tc_tiling_on_sc=False),
)
def _agg1_kernel(src2, dst2, xs0, xs1, t0, t1,
                 sidx, didx, rows, stage, accum, semg, sems):
    c = lax.axis_index("c")
    s = lax.axis_index("s")

    @pl.when(c == 0)
    def _():
        _agg_pass(src2, dst2, xs0, t0, sidx, didx, rows, stage, accum,
                  semg, sems, s)

    @pl.when(c == 1)
    def _():
        _agg_pass(src2, dst2, xs1, t1, sidx, didx, rows, stage, accum,
                  semg, sems, s)


@functools.partial(
    pl.kernel,
    out_type=tuple(jax.ShapeDtypeStruct((NP, L), _f32) for _ in range(4)),
    mesh=plsc.VectorSubcoreMesh(core_axis_name="c", subcore_axis_name="s"),
    scratch_types=[
        pltpu.VMEM((KSUB, 128), _i32),
        pltpu.VMEM((KSUB, 128), _i32),
        pltpu.VMEM((CH, L), _f32),
        pltpu.VMEM((STG, L), _f32),
        pltpu.VMEM_SHARED((NP, L), _f32),
        pltpu.SemaphoreType.DMA,
        pltpu.SemaphoreType.DMA,
    ],
    compiler_params=pltpu.CompilerParams(use_tc_tiling_on_sc=False),
)
def _agg2_kernel(src2, dst2, g0, g1, g2, g3, u0, u1, u2, u3,
                 sidx, didx, rows, stage, accum, semg, sems):
    c = lax.axis_index("c")
    s = lax.axis_index("s")

    @pl.when(c == 0)
    def _():
        for tbl, out in ((g0, u0), (g1, u1)):
            _agg_pass(src2, dst2, tbl, out, sidx, didx, rows, stage, accum,
                      semg, sems, s)

    @pl.when(c == 1)
    def _():
        for tbl, out in ((g2, u2), (g3, u3)):
            _agg_pass(src2, dst2, tbl, out, sidx, didx, rows, stage, accum,
                      semg, sems, s)


NPR = NP // 8
BP = BN // 8
NSEG = G + 1
PNT = NP // 32
PNR = PNT // 8


@functools.partial(
    pl.kernel,
    out_type=(jax.ShapeDtypeStruct((32, NSEG, L), _f32),
              jax.ShapeDtypeStruct((32, NSEG, L), _f32)),
    mesh=plsc.VectorSubcoreMesh(core_axis_name="c", subcore_axis_name="s"),
    scratch_types=[
        pltpu.VMEM((PNR, 128), _f32),
        pltpu.VMEM((PNT,), _i32),
        pltpu.VMEM((NSEG, L), _f32),
        pltpu.VMEM((NSEG, L), _f32),
    ],
    compiler_params=pltpu.CompilerParams(use_tc_tiling_on_sc=False,
                                         needs_layout_passes=False),
)
def _pool_kernel(zp, bpad, sums_o, cnts_o, zv, bv, sacc, cacc):
    c = lax.axis_index("c")
    s = lax.axis_index("s")
    w = c * NTILES + s
    _fill(sacc, NSEG, jnp.zeros((L,), _f32))
    _fill(cacc, NSEG, jnp.zeros((L,), _f32))
    pltpu.sync_copy(zp.at[pl.ds(w * PNR, PNR)], zv)
    pltpu.sync_copy(bpad.at[pl.ds(w * PNT, PNT)], bv)
    lanes = lax.iota(_i32, L)
    ones16 = jnp.ones((L,), _f32)

    def grp(g, _):
        nloc = g * L + lanes
        ridx = nloc >> 3
        lidx = (nloc & 7) * L
        z16 = plsc.load_gather(zv, [ridx, lidx])
        b16 = bv[pl.ds(g * L, L)]
        plsc.addupdate_scatter(sacc, [b16, lanes], z16)
        plsc.addupdate_scatter(cacc, [b16, lanes], ones16)
        return 0
    lax.fori_loop(0, PNT // L, grp, 0)
    pltpu.sync_copy(sacc, sums_o.at[w])
    pltpu.sync_copy(cacc, cnts_o.at[w])


def _prep_body(p0, p1, xp0, xp1, dinv_o, xs0_o, xs1_o):
    dinvp = lax.rsqrt(1.0 + p0[...] + p1[...])
    dinv_o[...] = dinvp
    xs0_o[...] = xp0[...] * dinvp
    xs1_o[...] = xp1[...] * dinvp


def _layer1_body(t0, t1, xs0, xs1, dinv, m1, b1p, g0, g1, g2, g3):
    dinvp = dinv[...]
    a0 = (t0[...] + xs0[...]) * dinvp
    a1 = (t1[...] + xs1[...]) * dinvp
    m = m1[...]
    b = b1p[...]
    for k, out in enumerate((g0, g1, g2, g3)):
        h = (jnp.dot(a0, m[0, k], preferred_element_type=_f32)
             + jnp.dot(a1, m[1, k], preferred_element_type=_f32)
             + b[k:k + 1, :])
        out[...] = jnp.maximum(h, 0.0) * dinvp


def _layer2_body(u0, u1, u2, u3, g0, g1, g2, g3, dinv, m2, b2p, mz, z_o):
    dinvp = dinv[...]
    aggs = [(u[...] + g[...]) * dinvp
            for u, g in ((u0, g0), (u1, g1), (u2, g2), (u3, g3))]
    m = m2[...]
    b = b2p[...]
    mzv = mz[...]
    zp = jnp.zeros_like(dinvp)
    for k in range(4):
        h = b[k:k + 1, :]
        for s in range(4):
            h = h + jnp.dot(aggs[s], m[s, k], preferred_element_type=_f32)
        h = jnp.maximum(h, 0.0)
        zp = zp + jnp.dot(h, mzv[k], preferred_element_type=_f32)
    z_o[...] = zp


def _combine_body(sums, cnts, lb, out_o):
    ssum = jnp.sum(sums[...][:G, :], axis=1, keepdims=True)
    csum = jnp.sum(cnts[...][:G, :], axis=1, keepdims=True)
    out_o[...] = ssum / jnp.maximum(csum, 1.0) + lb[...]


def _pk_spec():
    return pl.BlockSpec((BP, 128), lambda i: (i, 0))


def kernel(x, edge_index, batch, W1, b1, W2, b2, lin_W, lin_b):
    pad_row = jnp.arange(128, dtype=_i32)
    npad = (EP - E) // 128
    src2 = jnp.concatenate(
        [edge_index[0].reshape(E // 128, 128),
         jnp.broadcast_to(pad_row, (npad, 128))])
    dst2 = jnp.concatenate(
        [edge_index[1].reshape(E // 128, 128),
         jnp.broadcast_to(N + pad_row, (npad, 128))])
    bpad = jnp.concatenate([batch, jnp.full((NP - N,), G, _i32)])

    eye8 = jnp.eye(8, dtype=_f32)

    def bd(w16):
        return jnp.einsum("pq,ij->piqj", eye8, w16).reshape(128, 128)

    m1 = jnp.stack([jnp.stack([bd(W1[16 * s:16 * s + 16, 16 * k:16 * k + 16])
                               for k in range(4)]) for s in range(2)])
    m2 = jnp.stack([jnp.stack([bd(W2[16 * s:16 * s + 16, 16 * k:16 * k + 16])
                               for k in range(4)]) for s in range(4)])
    mz = jnp.stack([jnp.einsum("pq,i,j->piqj", eye8,
                               lin_W[16 * k:16 * k + 16, 0],
                               jnp.ones((L,), _f32)).reshape(128, 128)
                    for k in range(4)])
    b1p = jnp.broadcast_to(b1.reshape(4, 1, L), (4, 8, L)).reshape(4, 128)
    b2p = jnp.broadcast_to(b2.reshape(4, 1, L), (4, 8, L)).reshape(4, 128)

    p0, p1 = _deg_kernel(dst2)

    grid = (NP // BN,)
    xpad = jnp.concatenate([x, jnp.zeros((NP - N, IN), _f32)])
    xp0 = xpad[:, :L].reshape(NPR, 128)
    xp1 = xpad[:, L:].reshape(NPR, 128)
    dinvp, xs0p, xs1p = pl.pallas_call(
        _prep_body, grid=grid,
        in_specs=[_pk_spec()] * 4,
        out_specs=[_pk_spec()] * 3,
        out_shape=[jax.ShapeDtypeStruct((NPR, 128), _f32)] * 3,
    )(p0.reshape(NPR, 128), p1.reshape(NPR, 128), xp0, xp1)

    t0, t1 = _agg1_kernel(src2, dst2, xs0p.reshape(NP, L), xs1p.reshape(NP, L))

    g0p, g1p, g2p, g3p = pl.pallas_call(
        _layer1_body, grid=grid,
        in_specs=[_pk_spec()] * 5 + [
            pl.BlockSpec((2, 4, 128, 128), lambda i: (0, 0, 0, 0)),
            pl.BlockSpec((4, 128), lambda i: (0, 0))],
        out_specs=[_pk_spec()] * 4,
        out_shape=[jax.ShapeDtypeStruct((NPR, 128), _f32)] * 4,
    )(t0.reshape(NPR, 128), t1.reshape(NPR, 128), xs0p, xs1p, dinvp, m1, b1p)

    u0, u1, u2, u3 = _agg2_kernel(src2, dst2, g0p.reshape(NP, L),
                                  g1p.reshape(NP, L), g2p.reshape(NP, L),
                                  g3p.reshape(NP, L))

    zp = pl.pallas_call(
        _layer2_body, grid=grid,
        in_specs=[_pk_spec()] * 9 + [
            pl.BlockSpec((4, 4, 128, 128), lambda i: (0, 0, 0, 0)),
            pl.BlockSpec((4, 128), lambda i: (0, 0)),
            pl.BlockSpec((4, 128, 128), lambda i: (0, 0, 0))],
        out_specs=_pk_spec(),
        out_shape=jax.ShapeDtypeStruct((NPR, 128), _f32),
    )(u0.reshape(NPR, 128), u1.reshape(NPR, 128), u2.reshape(NPR, 128),
      u3.reshape(NPR, 128), g0p, g1p, g2p, g3p, dinvp, m2, b2p, mz)

    sums, cnts = _pool_kernel(zp, bpad)
    sums = sums.transpose(1, 0, 2).reshape(NSEG, 512)
    cnts = cnts.transpose(1, 0, 2).reshape(NSEG, 512)

    out = pl.pallas_call(
        _combine_body, grid=(1,),
        in_specs=[pl.BlockSpec((NSEG, 512), lambda i: (0, 0)),
                  pl.BlockSpec((NSEG, 512), lambda i: (0, 0)),
                  pl.BlockSpec((1, 1), lambda i: (0, 0))],
        out_specs=pl.BlockSpec((G, 1), lambda i: (0, 0)),
        out_shape=jax.ShapeDtypeStruct((G, 1), _f32),
    )(sums, cnts, lin_b.reshape(1, 1))
    return out


# speedup vs baseline: 40.4627x; 1.1767x over previous
import functools

import jax
import jax.numpy as jnp
from jax import lax
from jax.experimental import pallas as pl
from jax.experimental.pallas import tpu as pltpu
from jax.experimental.pallas import tpu_sc as plsc

N = 100000
E = 1600000
IN = 32
H = 64
G = 64

L = 16
F = 32
NTILES = 16
BN = 2048
NP = 49 * BN
NP4 = NP // 4
BP4 = BN // 4
EP = 1638400
CH = 1280
KSUB = CH // 128
RPT = NP // NTILES
STG = RPT // 16
NSEG = G + 1
PNT = NP // 32
PNR4 = PNT // 4

_f32 = jnp.float32
_bf16 = jnp.bfloat16
_i32 = jnp.int32


def _fill(ref, rows, vec):
    def body(i, _):
        ref[i] = vec
        return 0
    lax.fori_loop(0, rows, body, 0)


def _zero_accum(stage, accum, s):
    _fill(stage, STG, jnp.zeros((F,), _bf16))
    rz = s * RPT
    for r in range(16):
        pltpu.sync_copy(stage, accum.at[pl.ds(rz + r * STG, STG)])


def _write_out(stage, accum, out, s):
    rz = s * RPT
    for r in range(16):
        pltpu.sync_copy(accum.at[pl.ds(rz + r * STG, STG)], stage)
        pltpu.sync_copy(stage, out.at[pl.ds(rz + r * STG, STG)])


def _scan_edges(src2, dst2, table, sidx, didx, rows, accum,
                semg, sems, wid, nch):
    def chunk(j, _):
        rb = wid * (nch * KSUB) + j * KSUB
        pltpu.sync_copy(src2.at[pl.ds(rb, KSUB)], sidx)
        pltpu.sync_copy(dst2.at[pl.ds(rb, KSUB)], didx)
        gd = [pltpu.async_copy(table.at[sidx.at[k]],
                               rows.at[pl.ds(k * 128, 128)], semg)
              for k in range(KSUB)]
        sd = []
        for k in range(KSUB):
            gd[k].wait()
            sd.append(pltpu.async_copy(rows.at[pl.ds(k * 128, 128)],
                                       accum.at[didx.at[k]], sems, add=True))
        for d in sd:
            d.wait()
        return 0
    lax.fori_loop(0, nch, chunk, 0)


def _agg_pass(src2, dst2, table, out, sidx, didx, rows, stage, accum,
              semg, sems, s, wid, nch):
    _zero_accum(stage, accum, s)
    plsc.subcore_barrier()
    _scan_edges(src2, dst2, table, sidx, didx, rows, accum,
                semg, sems, wid, nch)
    plsc.subcore_barrier()
    _write_out(stage, accum, out, s)
    plsc.subcore_barrier()


_AGG_SCRATCH = [
    pltpu.VMEM((KSUB, 128), _i32),
    pltpu.VMEM((KSUB, 128), _i32),
    pltpu.VMEM((CH, F), _bf16),
    pltpu.VMEM((STG, F), _bf16),
    pltpu.VMEM_SHARED((NP, F), _bf16),
    pltpu.SemaphoreType.DMA,
    pltpu.SemaphoreType.DMA,
]


@functools.partial(
    pl.kernel,
    out_type=(jax.ShapeDtypeStruct((NP, F), _bf16),
              jax.ShapeDtypeStruct((NP, F), _bf16)),
    mesh=plsc.VectorSubcoreMesh(core_axis_name="c", subcore_axis_name="s"),
    scratch_types=[
        pltpu.VMEM((KSUB, 128), _i32),
        pltpu.VMEM((128, F), _bf16),
        pltpu.VMEM((STG, F), _bf16),
        pltpu.VMEM_SHARED((NP, F), _bf16),
        pltpu.SemaphoreType.DMA,
    ],
    compiler_params=pltpu.CompilerParams(use_tc_tiling_on_sc=False),
)
def _deg_kernel(dst2, o0, o1, didx, ones_v, stage, accum, sem):
    c = lax.axis_index("c")
    s = lax.axis_index("s")
    w = c * NTILES + s
    _fill(ones_v, 128, jnp.ones((F,), _bf16))
    _zero_accum(stage, accum, s)
    plsc.subcore_barrier()

    def chunk(j, _):
        rb = w * (EP // 32 // 128) + j * KSUB
        pltpu.sync_copy(dst2.at[pl.ds(rb, KSUB)], didx)
        sd = [pltpu.async_copy(ones_v, accum.at[didx.at[k]], sem, add=True)
              for k in range(KSUB)]
        for d in sd:
            d.wait()
        return 0
    lax.fori_loop(0, EP // 32 // CH, chunk, 0)
    plsc.subcore_barrier()

    @pl.when(c == 0)
    def _():
        _write_out(stage, accum, o0, s)

    @pl.when(c == 1)
    def _():
        _write_out(stage, accum, o1, s)


@functools.partial(
    pl.kernel,
    out_type=(jax.ShapeDtypeStruct((NP, F), _bf16),
              jax.ShapeDtypeStruct((NP, F), _bf16)),
    mesh=plsc.VectorSubcoreMesh(core_axis_name="c", subcore_axis_name="s"),
    scratch_types=list(_AGG_SCRATCH),
    compiler_params=pltpu.CompilerParams(use_tc_tiling_on_sc=False),
)
def _agg1_kernel(src2, dst2, xsb, t0, t1,
                 sidx, didx, rows, stage, accum, semg, sems):
    c = lax.axis_index("c")
    s = lax.axis_index("s")
    w = c * NTILES + s
    _zero_accum(stage, accum, s)
    plsc.subcore_barrier()
    _scan_edges(src2, dst2, xsb, sidx, didx, rows, accum,
                semg, sems, w, EP // 32 // CH)
    plsc.subcore_barrier()

    @pl.when(c == 0)
    def _():
        _write_out(stage, accum, t0, s)

    @pl.when(c == 1)
    def _():
        _write_out(stage, accum, t1, s)


@functools.partial(
    pl.kernel,
    out_type=(jax.ShapeDtypeStruct((NP, F), _bf16),
              jax.ShapeDtypeStruct((NP, F), _bf16)),
    mesh=plsc.VectorSubcoreMesh(core_axis_name="c", subcore_axis_name="s"),
    scratch_types=list(_AGG_SCRATCH),
    compiler_params=pltpu.CompilerParams(use_tc_tiling_on_sc=False),
)
def _agg2_kernel(src2, dst2, g0, g1, u0, u1,
                 sidx, didx, rows, stage, accum, semg, sems):
    c = lax.axis_index("c")
    s = lax.axis_index("s")

    @pl.when(c == 0)
    def _():
        _agg_pass(src2, dst2, g0, u0, sidx, didx, rows, stage, accum,
                  semg, sems, s, s, EP // NTILES // CH)

    @pl.when(c == 1)
    def _():
        _agg_pass(src2, dst2, g1, u1, sidx, didx, rows, stage, accum,
                  semg, sems, s, s, EP // NTILES // CH)


@functools.partial(
    pl.kernel,
    out_type=(jax.ShapeDtypeStruct((32, NSEG, L), _f32),
              jax.ShapeDtypeStruct((32, NSEG, L), _f32)),
    mesh=plsc.VectorSubcoreMesh(core_axis_name="c", subcore_axis_name="s"),
    scratch_types=[
        pltpu.VMEM((PNR4, 128), _f32),
        pltpu.VMEM((PNT,), _i32),
        pltpu.VMEM((NSEG, L), _f32),
        pltpu.VMEM((NSEG, L), _f32),
    ],
    compiler_params=pltpu.CompilerParams(use_tc_tiling_on_sc=False,
                                         needs_layout_passes=False),
)
def _pool_kernel(zp, bpad, sums_o, cnts_o, zv, bv, sacc, cacc):
    c = lax.axis_index("c")
    s = lax.axis_index("s")
    w = c * NTILES + s
    _fill(sacc, NSEG, jnp.zeros((L,), _f32))
    _fill(cacc, NSEG, jnp.zeros((L,), _f32))
    pltpu.sync_copy(zp.at[pl.ds(w * PNR4, PNR4)], zv)
    pltpu.sync_copy(bpad.at[pl.ds(w * PNT, PNT)], bv)
    lanes = lax.iota(_i32, L)
    ones16 = jnp.ones((L,), _f32)

    def grp(g, _):
        nloc = g * L + lanes
        ridx = nloc >> 2
        lidx = (nloc & 3) * F
        z16 = plsc.load_gather(zv, [ridx, lidx])
        b16 = bv[pl.ds(g * L, L)]
        plsc.addupdate_scatter(sacc, [b16, lanes], z16)
        plsc.addupdate_scatter(cacc, [b16, lanes], ones16)
        return 0
    lax.fori_loop(0, PNT // L, grp, 0)
    pltpu.sync_copy(sacc, sums_o.at[w])
    pltpu.sync_copy(cacc, cnts_o.at[w])


def _prep_body(p0, p1, x4, dinv_o, xsb_o):
    deg = 1.0 + p0[...].astype(_f32) + p1[...].astype(_f32)
    dinv4 = lax.rsqrt(deg)
    dinv_o[...] = dinv4
    xsb_o[...] = (x4[...] * dinv4).astype(_bf16)


def _layer1_body(tp0, tp1, xsb, dinv, m1b, b1p, g0_o, g1_o):
    dinv4 = dinv[...]
    t = tp0[...].astype(_f32) + tp1[...].astype(_f32)
    agg = (t + xsb[...].astype(_f32)) * dinv4
    m = m1b[...]
    b = b1p[...]
    for k, out in enumerate((g0_o, g1_o)):
        h = jnp.dot(agg, m[k], preferred_element_type=_f32) + b[k:k + 1, :]
        out[...] = (jnp.maximum(h, 0.0) * dinv4).astype(_bf16)


def _layer2_body(u0, u1, g0, g1, dinv, m2b, b2p, mzb, z_o):
    dinv4 = dinv[...]
    aggs = [(u[...].astype(_f32) + g[...].astype(_f32)) * dinv4
            for u, g in ((u0, g0), (u1, g1))]
    m = m2b[...]
    b = b2p[...]
    mzv = mzb[...]
    zp = jnp.zeros((BP4, 128), _f32)
    for k in range(2):
        h = b[k:k + 1, :]
        for s in range(2):
            h = h + jnp.dot(aggs[s], m[s, k], preferred_element_type=_f32)
        h = jnp.maximum(h, 0.0)
        zp = zp + jnp.dot(h, mzv[k], preferred_element_type=_f32)
    z_o[...] = zp


def _combine_body(sums, cnts, lb, out_o):
    ssum = jnp.sum(sums[...][:G, :], axis=1, keepdims=True)
    csum = jnp.sum(cnts[...][:G, :], axis=1, keepdims=True)
    out_o[...] = ssum / jnp.maximum(csum, 1.0) + lb[...]


def _pk_spec():
    return pl.BlockSpec((BP4, 128), lambda i: (i, 0))


def kernel(x, edge_index, batch, W1, b1, W2, b2, lin_W, lin_b):
    pad_row = jnp.arange(128, dtype=_i32)
    npad = (EP - E) // 128
    src2 = jnp.concatenate(
        [edge_index[0].reshape(E // 128, 128),
         jnp.broadcast_to(pad_row, (npad, 128))])
    dst2 = jnp.concatenate(
        [edge_index[1].reshape(E // 128, 128),
         jnp.broadcast_to(N + pad_row, (npad, 128))])
    bpad = jnp.concatenate([batch, jnp.full((NP - N,), G, _i32)])
    x4 = jnp.concatenate([x, jnp.zeros((NP - N, IN), _f32)]).reshape(NP4, 128)

    eye4 = jnp.eye(4, dtype=_f32)

    def bd4(w32):
        return jnp.einsum("pq,ij->piqj", eye4, w32).reshape(128, 128)

    m1b = jnp.stack([bd4(W1[:, 32 * k:32 * k + 32]) for k in range(2)])
    m2b = jnp.stack([jnp.stack([bd4(W2[32 * s:32 * s + 32, 32 * k:32 * k + 32])
                                for k in range(2)]) for s in range(2)])
    mzb = jnp.stack([jnp.einsum("pq,i,j->piqj", eye4,
                                lin_W[32 * k:32 * k + 32, 0],
                                jnp.ones((F,), _f32)).reshape(128, 128)
                     for k in range(2)])
    b1p = jnp.broadcast_to(b1.reshape(2, 1, F), (2, 4, F)).reshape(2, 128)
    b2p = jnp.broadcast_to(b2.reshape(2, 1, F), (2, 4, F)).reshape(2, 128)

    p0, p1 = _deg_kernel(dst2)

    grid = (NP4 // BP4,)
    dinv4, xsb = pl.pallas_call(
        _prep_body, grid=grid,
        in_specs=[_pk_spec()] * 3,
        out_specs=[_pk_spec()] * 2,
        out_shape=[jax.ShapeDtypeStruct((NP4, 128), _f32),
                   jax.ShapeDtypeStruct((NP4, 128), _bf16)],
    )(p0.reshape(NP4, 128), p1.reshape(NP4, 128), x4)

    tp0, tp1 = _agg1_kernel(src2, dst2, xsb.reshape(NP, F))

    g0b, g1b = pl.pallas_call(
        _layer1_body, grid=grid,
        in_specs=[_pk_spec()] * 4 + [
            pl.BlockSpec((2, 128, 128), lambda i: (0, 0, 0)),
            pl.BlockSpec((2, 128), lambda i: (0, 0))],
        out_specs=[_pk_spec()] * 2,
        out_shape=[jax.ShapeDtypeStruct((NP4, 128), _bf16)] * 2,
    )(tp0.reshape(NP4, 128), tp1.reshape(NP4, 128), xsb, dinv4, m1b, b1p)

    u0b, u1b = _agg2_kernel(src2, dst2, g0b.reshape(NP, F), g1b.reshape(NP, F))

    zp4 = pl.pallas_call(
        _layer2_body, grid=grid,
        in_specs=[_pk_spec()] * 5 + [
            pl.BlockSpec((2, 2, 128, 128), lambda i: (0, 0, 0, 0)),
            pl.BlockSpec((2, 128), lambda i: (0, 0)),
            pl.BlockSpec((2, 128, 128), lambda i: (0, 0, 0))],
        out_specs=_pk_spec(),
        out_shape=jax.ShapeDtypeStruct((NP4, 128), _f32),
    )(u0b.reshape(NP4, 128), u1b.reshape(NP4, 128), g0b, g1b, dinv4,
      m2b, b2p, mzb)

    sums, cnts = _pool_kernel(zp4, bpad)
    sums = sums.transpose(1, 0, 2).reshape(NSEG, 512)
    cnts = cnts.transpose(1, 0, 2).reshape(NSEG, 512)

    out = pl.pallas_call(
        _combine_body, grid=(1,),
        in_specs=[pl.BlockSpec((NSEG, 512), lambda i: (0, 0)),
                  pl.BlockSpec((NSEG, 512), lambda i: (0, 0)),
                  pl.BlockSpec((1, 1), lambda i: (0, 0))],
        out_specs=pl.BlockSpec((G, 1), lambda i: (0, 0)),
        out_shape=jax.ShapeDtypeStruct((G, 1), _f32),
    )(sums, cnts, lin_b.reshape(1, 1))
    return out
